# Initial kernel scaffold; baseline (speedup 1.0000x reference)
import jax
import jax.numpy as jnp
from jax.experimental import pallas as pl


def kernel(x, pos_edge_index, neg_edge_index, W_pos_l, W_pos_r, b_pos, W_neg_l, W_neg_r, b_neg, gamma, beta):
    raise NotImplementedError("write your pallas kernel here")


# speedup vs baseline: 5.9303x; 5.9303x over previous
import functools

import jax
import jax.numpy as jnp
from jax import lax
from jax.experimental import pallas as pl
from jax.experimental.pallas import tpu as pltpu
from jax.experimental.pallas import tpu_sc as plsc

N = 10000
E = 320000
D_IN = 128
D_OUT = 64
EPS = 1e-5

NS = 16
EPT = E // NS
CH = 80
NCH = EPT // CH
NPAD = 10240
RPT = NPAD // NS
RB = 40


def _mm_l_body(x_ref, w_ref, h_ref):
    h = jnp.dot(x_ref[...], w_ref[...], preferred_element_type=jnp.float32)
    h_ref[0:N, :] = h[:, 0:D_OUT]
    h_ref[N:2 * N, :] = h[:, D_OUT:2 * D_OUT]


def _mm_l(x, w_l):
    return pl.pallas_call(
        _mm_l_body,
        out_shape=jax.ShapeDtypeStruct((2 * N, D_OUT), jnp.float32),
    )(x, w_l)


def _sc_body(h_hbm, src_hbm, dst_hbm, sum_hbm, cnt_hbm,
             sidx, didx, rows, ones, zb64, zb16, buf64, buf16,
             accum_sh, cnt_sh, sem):
    c = lax.axis_index("c")
    s = lax.axis_index("s")

    zero16 = jnp.zeros((16,), jnp.float32)
    one16 = jnp.ones((16,), jnp.float32)
    for i in range(RB):
        for j in range(4):
            zb64[i, pl.ds(j * 16, 16)] = zero16
        zb16[i, :] = zero16
    for i in range(CH):
        ones[i, :] = one16

    def zloop(i, _):
        r0 = s * RPT + i * RB
        pltpu.sync_copy(zb64, accum_sh.at[pl.ds(r0, RB)])
        pltpu.sync_copy(zb16, cnt_sh.at[pl.ds(r0, RB)])
        return 0

    lax.fori_loop(0, RPT // RB, zloop, 0)
    plsc.subcore_barrier()

    ebase = c * E + s * EPT
    hoff = c * N

    def eloop(i, _):
        b = ebase + i * CH
        pltpu.sync_copy(src_hbm.at[pl.ds(b, CH)], sidx)
        pltpu.sync_copy(dst_hbm.at[pl.ds(b, CH)], didx)
        for j in range(CH // 16):
            sidx[pl.ds(j * 16, 16)] = sidx[pl.ds(j * 16, 16)] + hoff
        pltpu.async_copy(h_hbm.at[sidx], rows, sem).wait()
        pltpu.sync_copy(rows, accum_sh.at[didx], add=True)
        pltpu.sync_copy(ones, cnt_sh.at[didx], add=True)
        return 0

    lax.fori_loop(0, NCH, eloop, 0)
    plsc.subcore_barrier()

    def wloop(i, _):
        r0 = s * RPT + i * RB
        g0 = c * NPAD + r0
        pltpu.sync_copy(accum_sh.at[pl.ds(r0, RB)], buf64)
        pltpu.sync_copy(buf64, sum_hbm.at[pl.ds(g0, RB)])
        pltpu.sync_copy(cnt_sh.at[pl.ds(r0, RB)], buf16)
        pltpu.sync_copy(buf16, cnt_hbm.at[pl.ds(g0, RB)])
        return 0

    lax.fori_loop(0, RPT // RB, wloop, 0)


_sc_aggregate = functools.partial(
    pl.kernel,
    out_type=(
        jax.ShapeDtypeStruct((2 * NPAD, D_OUT), jnp.float32),
        jax.ShapeDtypeStruct((2 * NPAD, 16), jnp.float32),
    ),
    mesh=plsc.VectorSubcoreMesh(core_axis_name="c", subcore_axis_name="s"),
    compiler_params=pltpu.CompilerParams(use_tc_tiling_on_sc=False),
    scratch_types=(
        pltpu.VMEM((CH,), jnp.int32),
        pltpu.VMEM((CH,), jnp.int32),
        pltpu.VMEM((CH, D_OUT), jnp.float32),
        pltpu.VMEM((CH, 16), jnp.float32),
        pltpu.VMEM((RB, D_OUT), jnp.float32),
        pltpu.VMEM((RB, 16), jnp.float32),
        pltpu.VMEM((RB, D_OUT), jnp.float32),
        pltpu.VMEM((RB, 16), jnp.float32),
        pltpu.VMEM_SHARED((NPAD, D_OUT), jnp.float32),
        pltpu.VMEM_SHARED((NPAD, 16), jnp.float32),
        pltpu.SemaphoreType.DMA,
    ),
)(_sc_body)


def _finish_body(x_ref, wr_ref, sum_ref, cnt_ref, b_ref, g_ref, be_ref,
                 out_ref):
    r = jnp.dot(x_ref[...], wr_ref[...], preferred_element_type=jnp.float32)
    cp = jnp.maximum(cnt_ref[0:N, 0:1], 1.0)
    cn = jnp.maximum(cnt_ref[NPAD:NPAD + N, 0:1], 1.0)
    yp = sum_ref[0:N, :] / cp
    yn = sum_ref[NPAD:NPAD + N, :] / cn
    y = jnp.concatenate([yp, yn], axis=1) + r + b_ref[...]
    mu = jnp.mean(y, axis=0, keepdims=True)
    var = jnp.mean((y - mu) ** 2, axis=0, keepdims=True)
    out = (y - mu) * jax.lax.rsqrt(var + EPS) * g_ref[...] + be_ref[...]
    out_ref[...] = jnp.maximum(out, 0.0)


def _finish(x, wr, sums, cnts, b, g, be):
    return pl.pallas_call(
        _finish_body,
        out_shape=jax.ShapeDtypeStruct((N, 2 * D_OUT), jnp.float32),
    )(x, wr, sums, cnts, b, g, be)


def kernel(x, pos_edge_index, neg_edge_index, W_pos_l, W_pos_r, b_pos,
           W_neg_l, W_neg_r, b_neg, gamma, beta):
    w_l = jnp.concatenate([W_pos_l, W_neg_l], axis=1)
    w_r = jnp.concatenate([W_pos_r, W_neg_r], axis=1)
    src = jnp.concatenate([pos_edge_index[0], neg_edge_index[0]]
                          ).astype(jnp.int32)
    dst = jnp.concatenate([pos_edge_index[1], neg_edge_index[1]]
                          ).astype(jnp.int32)
    bias = jnp.concatenate([b_pos, b_neg]).reshape(1, 2 * D_OUT)
    g = gamma.reshape(1, 2 * D_OUT)
    be = beta.reshape(1, 2 * D_OUT)

    h = _mm_l(x, w_l)
    sums, cnts = _sc_aggregate(h, src, dst)
    return _finish(x, w_r, sums, cnts, bias, g, be)


# speedup vs baseline: 16.2127x; 2.7339x over previous
import functools

import jax
import jax.numpy as jnp
from jax import lax
from jax.experimental import pallas as pl
from jax.experimental.pallas import tpu as pltpu
from jax.experimental.pallas import tpu_sc as plsc

N = 10000
E = 320000
D_IN = 128
D_OUT = 64
EPS = 1e-5

NS = 16
EPT = E // NS
CH = 80
NCH = EPT // CH
NBUF = 5
NPAD = 10240
RPT = NPAD // NS
RB = 40


def _mm_l_body(x_ref, w_ref, h_ref):
    h = jnp.dot(x_ref[...], w_ref[...], preferred_element_type=jnp.float32)
    h_ref[0:N, :] = h[:, 0:D_OUT]
    h_ref[N:2 * N, :] = h[:, D_OUT:2 * D_OUT]


def _mm_l(x, w_l):
    return pl.pallas_call(
        _mm_l_body,
        out_shape=jax.ShapeDtypeStruct((2 * N, D_OUT), jnp.float32),
    )(x, w_l)


def _sc_body(h_hbm, src_hbm, dst_hbm, sum_hbm, cnt_hbm,
             sidx, didx, rows, ones, zb64, zb16, buf64, buf16,
             accum_sh, cnt_sh, gsem0, gsem1, gsem2, gsem3, gsem4, osem):
    c = lax.axis_index("c")
    s = lax.axis_index("s")
    gsems = (gsem0, gsem1, gsem2, gsem3, gsem4)

    zero16 = jnp.zeros((16,), jnp.float32)
    one16 = jnp.ones((16,), jnp.float32)
    for i in range(RB):
        for j in range(4):
            zb64[i, pl.ds(j * 16, 16)] = zero16
        zb16[i, :] = zero16
    for i in range(CH):
        ones[i, :] = one16

    def zloop(i, _):
        r0 = s * RPT + i * RB
        pltpu.sync_copy(zb64, accum_sh.at[pl.ds(r0, RB)])
        pltpu.sync_copy(zb16, cnt_sh.at[pl.ds(r0, RB)])
        return 0

    lax.fori_loop(0, RPT // RB, zloop, 0)
    plsc.subcore_barrier()

    cb = (c * NS + s) * NCH
    pltpu.sync_copy(src_hbm.at[pl.ds(cb, NCH)], sidx)
    pltpu.sync_copy(dst_hbm.at[pl.ds(cb, NCH)], didx)
    for b in range(NBUF):
        pltpu.async_copy(h_hbm.at[sidx.at[b]], rows.at[b], gsems[b])

    def outer(it, _):
        for b in range(NBUF):
            i = it * NBUF + b
            pltpu.make_async_copy(h_hbm.at[sidx.at[b]],
                                  rows.at[b], gsems[b]).wait()
            pltpu.sync_copy(rows.at[b], accum_sh.at[didx.at[i]], add=True)
            pltpu.async_copy(ones, cnt_sh.at[didx.at[i]], osem, add=True)
            inext = i + NBUF

            @pl.when(inext < NCH)
            def _():
                pltpu.async_copy(h_hbm.at[sidx.at[inext]], rows.at[b],
                                 gsems[b])
        return 0

    lax.fori_loop(0, NCH // NBUF, outer, 0)

    def drain(i, _):
        pltpu.make_async_copy(ones, cnt_sh.at[didx.at[0]], osem).wait()
        return 0

    lax.fori_loop(0, NCH, drain, 0)
    plsc.subcore_barrier()

    def wloop(i, _):
        r0 = s * RPT + i * RB
        g0 = c * NPAD + r0
        pltpu.sync_copy(accum_sh.at[pl.ds(r0, RB)], buf64)
        pltpu.sync_copy(buf64, sum_hbm.at[pl.ds(g0, RB)])
        pltpu.sync_copy(cnt_sh.at[pl.ds(r0, RB)], buf16)
        pltpu.sync_copy(buf16, cnt_hbm.at[pl.ds(g0, RB)])
        return 0

    lax.fori_loop(0, RPT // RB, wloop, 0)


_sc_aggregate = functools.partial(
    pl.kernel,
    out_type=(
        jax.ShapeDtypeStruct((2 * NPAD, D_OUT), jnp.float32),
        jax.ShapeDtypeStruct((2 * NPAD, 16), jnp.float32),
    ),
    mesh=plsc.VectorSubcoreMesh(core_axis_name="c", subcore_axis_name="s"),
    compiler_params=pltpu.CompilerParams(use_tc_tiling_on_sc=False),
    scratch_types=(
        pltpu.VMEM((NCH, CH), jnp.int32),
        pltpu.VMEM((NCH, CH), jnp.int32),
        pltpu.VMEM((NBUF, CH, D_OUT), jnp.float32),
        pltpu.VMEM((CH, 16), jnp.float32),
        pltpu.VMEM((RB, D_OUT), jnp.float32),
        pltpu.VMEM((RB, 16), jnp.float32),
        pltpu.VMEM((RB, D_OUT), jnp.float32),
        pltpu.VMEM((RB, 16), jnp.float32),
        pltpu.VMEM_SHARED((NPAD, D_OUT), jnp.float32),
        pltpu.VMEM_SHARED((NPAD, 16), jnp.float32),
        pltpu.SemaphoreType.DMA,
        pltpu.SemaphoreType.DMA,
        pltpu.SemaphoreType.DMA,
        pltpu.SemaphoreType.DMA,
        pltpu.SemaphoreType.DMA,
        pltpu.SemaphoreType.DMA,
    ),
)(_sc_body)


def _finish_body(x_ref, wr_ref, sum_ref, cnt_ref, b_ref, g_ref, be_ref,
                 out_ref):
    r = jnp.dot(x_ref[...], wr_ref[...], preferred_element_type=jnp.float32)
    cp = jnp.maximum(cnt_ref[0:N, 0:1], 1.0)
    cn = jnp.maximum(cnt_ref[NPAD:NPAD + N, 0:1], 1.0)
    yp = sum_ref[0:N, :] / cp
    yn = sum_ref[NPAD:NPAD + N, :] / cn
    y = jnp.concatenate([yp, yn], axis=1) + r + b_ref[...]
    mu = jnp.mean(y, axis=0, keepdims=True)
    var = jnp.mean((y - mu) ** 2, axis=0, keepdims=True)
    out = (y - mu) * jax.lax.rsqrt(var + EPS) * g_ref[...] + be_ref[...]
    out_ref[...] = jnp.maximum(out, 0.0)


def _finish(x, wr, sums, cnts, b, g, be):
    return pl.pallas_call(
        _finish_body,
        out_shape=jax.ShapeDtypeStruct((N, 2 * D_OUT), jnp.float32),
    )(x, wr, sums, cnts, b, g, be)


def kernel(x, pos_edge_index, neg_edge_index, W_pos_l, W_pos_r, b_pos,
           W_neg_l, W_neg_r, b_neg, gamma, beta):
    w_l = jnp.concatenate([W_pos_l, W_neg_l], axis=1)
    w_r = jnp.concatenate([W_pos_r, W_neg_r], axis=1)
    src = jnp.concatenate([pos_edge_index[0].astype(jnp.int32),
                           neg_edge_index[0].astype(jnp.int32) + N]
                          ).reshape(2 * E // CH, CH)
    dst = jnp.concatenate([pos_edge_index[1], neg_edge_index[1]]
                          ).astype(jnp.int32).reshape(2 * E // CH, CH)
    bias = jnp.concatenate([b_pos, b_neg]).reshape(1, 2 * D_OUT)
    g = gamma.reshape(1, 2 * D_OUT)
    be = beta.reshape(1, 2 * D_OUT)

    h = _mm_l(x, w_l)
    sums, cnts = _sc_aggregate(h, src, dst)
    return _finish(x, w_r, sums, cnts, bias, g, be)


# speedup vs baseline: 17.1721x; 1.0592x over previous
import functools

import jax
import jax.numpy as jnp
from jax import lax
from jax.experimental import pallas as pl
from jax.experimental.pallas import tpu as pltpu
from jax.experimental.pallas import tpu_sc as plsc

N = 10000
E = 320000
D_IN = 128
D_OUT = 64
EPS = 1e-5

NS = 16
CH = 80
EPT = E // NS
NCH = EPT // CH
NBUF = 5
NPAD = 10240
RPT = NPAD // NS
RB = 40
ROWS_PER_SIGN = E // CH
DST_BASE = 2 * ROWS_PER_SIGN


def _mm_l_body(x_ref, w_ref, h_ref):
    h_ref[...] = jnp.dot(x_ref[...], w_ref[...],
                         preferred_element_type=jnp.float32)


def _mm_l(x, w_l):
    return pl.pallas_call(
        _mm_l_body,
        out_shape=jax.ShapeDtypeStruct((N, D_IN), jnp.float32),
    )(x, w_l)


def _sc_body(h_hbm, idx_hbm, mean_hbm,
             sidx, didx, rows, ones, zb64, zb16, buf64, cbuf,
             accum_sh, cnt_sh,
             gsem0, gsem1, gsem2, gsem3, gsem4,
             ssem0, ssem1, ssem2, ssem3, ssem4, osem):
    c = lax.axis_index("c")
    s = lax.axis_index("s")
    gsems = (gsem0, gsem1, gsem2, gsem3, gsem4)
    ssems = (ssem0, ssem1, ssem2, ssem3, ssem4)

    zero16 = jnp.zeros((16,), jnp.float32)
    one16 = jnp.ones((16,), jnp.float32)
    for i in range(RB):
        for j in range(4):
            zb64[i, pl.ds(j * 16, 16)] = zero16
        zb16[i, :] = zero16
    for i in range(CH):
        ones[i, :] = one16

    def zloop(i, _):
        r0 = s * RPT + i * RB
        pltpu.sync_copy(zb64, accum_sh.at[pl.ds(r0, RB)])
        pltpu.sync_copy(zb16, cnt_sh.at[pl.ds(r0, RB)])
        return 0

    lax.fori_loop(0, RPT // RB, zloop, 0)
    plsc.subcore_barrier()

    sbase = c * ROWS_PER_SIGN + s * NCH
    dbase = DST_BASE + c * ROWS_PER_SIGN + s * NCH
    pltpu.sync_copy(idx_hbm.at[pl.ds(sbase, NCH)], sidx)
    pltpu.sync_copy(idx_hbm.at[pl.ds(dbase, NCH)], didx)
    for b in range(NBUF):
        pltpu.async_copy(h_hbm.at[sidx.at[b]], rows.at[b], gsems[b])

    def outer(it, _):
        for b in range(NBUF):
            i = it * NBUF + b
            pltpu.make_async_copy(h_hbm.at[sidx.at[b]],
                                  rows.at[b], gsems[b]).wait()
            pltpu.async_copy(rows.at[b], accum_sh.at[didx.at[i]], ssems[b],
                             add=True)
            pltpu.async_copy(ones, cnt_sh.at[didx.at[i]], osem, add=True)
            pltpu.make_async_copy(rows.at[b], accum_sh.at[didx.at[i]],
                                  ssems[b]).wait()
            inext = i + NBUF

            @pl.when(inext < NCH)
            def _():
                pltpu.async_copy(h_hbm.at[sidx.at[inext]], rows.at[b],
                                 gsems[b])
        return 0

    lax.fori_loop(0, NCH // NBUF, outer, 0)

    def drain(i, _):
        pltpu.make_async_copy(ones, cnt_sh.at[didx.at[0]], osem).wait()
        return 0

    lax.fori_loop(0, NCH, drain, 0)
    plsc.subcore_barrier()

    def wloop(i, _):
        r0 = s * RPT + i * RB
        pltpu.sync_copy(accum_sh.at[pl.ds(r0, RB)], buf64)
        pltpu.sync_copy(cnt_sh.at[pl.ds(r0, RB)], cbuf)
        for r in range(RB):
            rec = 1.0 / jnp.maximum(cbuf[r, :], 1.0)
            for g in range(4):
                buf64[r, pl.ds(g * 16, 16)] = (
                    buf64[r, pl.ds(g * 16, 16)] * rec)
        pltpu.sync_copy(buf64, mean_hbm.at[pl.ds(c * NPAD + r0, RB)])
        return 0

    lax.fori_loop(0, RPT // RB, wloop, 0)


_sc_aggregate = functools.partial(
    pl.kernel,
    out_type=jax.ShapeDtypeStruct((2 * NPAD, D_OUT), jnp.float32),
    mesh=plsc.VectorSubcoreMesh(core_axis_name="c", subcore_axis_name="s"),
    compiler_params=pltpu.CompilerParams(use_tc_tiling_on_sc=False),
    scratch_types=(
        pltpu.VMEM((NCH, CH), jnp.int32),
        pltpu.VMEM((NCH, CH), jnp.int32),
        pltpu.VMEM((NBUF, CH, D_OUT), jnp.float32),
        pltpu.VMEM((CH, 16), jnp.float32),
        pltpu.VMEM((RB, D_OUT), jnp.float32),
        pltpu.VMEM((RB, 16), jnp.float32),
        pltpu.VMEM((RB, D_OUT), jnp.float32),
        pltpu.VMEM((RB, 16), jnp.float32),
        pltpu.VMEM_SHARED((NPAD, D_OUT), jnp.float32),
        pltpu.VMEM_SHARED((NPAD, 16), jnp.float32),
        pltpu.SemaphoreType.DMA,
        pltpu.SemaphoreType.DMA,
        pltpu.SemaphoreType.DMA,
        pltpu.SemaphoreType.DMA,
        pltpu.SemaphoreType.DMA,
        pltpu.SemaphoreType.DMA,
        pltpu.SemaphoreType.DMA,
        pltpu.SemaphoreType.DMA,
        pltpu.SemaphoreType.DMA,
        pltpu.SemaphoreType.DMA,
        pltpu.SemaphoreType.DMA,
    ),
)(_sc_body)


def _finish_body(x_ref, wr_ref, mean_ref, b_ref, g_ref, be_ref, out_ref):
    r = jnp.dot(x_ref[...], wr_ref[...], preferred_element_type=jnp.float32)
    m = jnp.concatenate([mean_ref[0:N, :], mean_ref[NPAD:NPAD + N, :]],
                        axis=1)
    y = m + r + b_ref[...]
    mu = jnp.mean(y, axis=0, keepdims=True)
    var = jnp.mean((y - mu) ** 2, axis=0, keepdims=True)
    out = (y - mu) * jax.lax.rsqrt(var + EPS) * g_ref[...] + be_ref[...]
    out_ref[...] = jnp.maximum(out, 0.0)


def _finish(x, wr, means, b, g, be):
    return pl.pallas_call(
        _finish_body,
        out_shape=jax.ShapeDtypeStruct((N, 2 * D_OUT), jnp.float32),
    )(x, wr, means, b, g, be)


def kernel(x, pos_edge_index, neg_edge_index, W_pos_l, W_pos_r, b_pos,
           W_neg_l, W_neg_r, b_neg, gamma, beta):
    w_l = jnp.concatenate([W_pos_l, W_neg_l], axis=1)
    w_r = jnp.concatenate([W_pos_r, W_neg_r], axis=1)
    idx_all = jnp.concatenate([
        pos_edge_index[0].astype(jnp.int32) * 2,
        neg_edge_index[0].astype(jnp.int32) * 2 + 1,
        pos_edge_index[1].astype(jnp.int32),
        neg_edge_index[1].astype(jnp.int32),
    ]).reshape(4 * ROWS_PER_SIGN, CH)
    bias = jnp.concatenate([b_pos, b_neg]).reshape(1, 2 * D_OUT)
    g = gamma.reshape(1, 2 * D_OUT)
    be = beta.reshape(1, 2 * D_OUT)

    h2 = _mm_l(x, w_l)
    h_flat = h2.reshape(2 * N, D_OUT)
    means = _sc_aggregate(h_flat, idx_all)
    return _finish(x, w_r, means, bias, g, be)
